# Initial kernel scaffold; baseline (speedup 1.0000x reference)
import jax
import jax.numpy as jnp
from jax.experimental import pallas as pl


def kernel(t_list, true_prob_r, edge_src, edge_dst, edge_rel, ent_embeds, rel_embeds, W_ih, W_hh, b_ih, b_hh, W_r, b_r):
    raise NotImplementedError("write your pallas kernel here")


# speedup vs baseline: 5.8039x; 5.8039x over previous
import functools

import jax
import jax.numpy as jnp
from jax import lax
from jax.experimental import pallas as pl
from jax.experimental.pallas import tpu as pltpu
from jax.experimental.pallas import tpu_sc as plsc

NC = 2
NS = 16
NW = NC * NS
LANES = 16
CHUNK = 128


def _sc_aggregate(esrc, edst, erel, ent_tab, rel_tab_hbm, S, B, H, EP, E):
  nchunk = EP // CHUNK
  nvec = H // LANES
  rel_rows = rel_tab_hbm.shape[0]
  inv_e = 1.0 / float(E)
  mesh = plsc.VectorSubcoreMesh(core_axis_name="c", subcore_axis_name="s")

  @functools.partial(
      pl.kernel,
      out_type=jax.ShapeDtypeStruct((S * B * 3 * H,), jnp.float32),
      mesh=mesh,
      scratch_types=[
          pltpu.VMEM((rel_rows, H), jnp.float32),
          pltpu.VMEM((CHUNK,), jnp.int32),
          pltpu.VMEM((CHUNK,), jnp.int32),
          pltpu.VMEM((CHUNK,), jnp.int32),
          pltpu.VMEM((CHUNK, H), jnp.float32),
          pltpu.VMEM((CHUNK, H), jnp.float32),
          pltpu.VMEM((S * 3 * H,), jnp.float32),
          pltpu.SemaphoreType.DMA,
          pltpu.SemaphoreType.DMA,
      ],
  )
  def k(esrc_hbm, edst_hbm, erel_hbm, ent_hbm, rel_hbm, out_hbm,
        rtab, sidx, didx, ridx, sbuf, dbuf, res, sem_s, sem_d):
    wid = lax.axis_index("s") * NC + lax.axis_index("c")
    pltpu.sync_copy(rel_hbm, rtab)
    ebase = wid * (S * EP)
    zvec = jnp.zeros((LANES,), jnp.float32)

    def seg_body(seg, carry):
      def chunk_body(c, acc):
        base = pl.multiple_of(ebase + seg * EP + c * CHUNK, CHUNK)
        pltpu.sync_copy(esrc_hbm.at[pl.ds(base, CHUNK)], sidx)
        pltpu.sync_copy(edst_hbm.at[pl.ds(base, CHUNK)], didx)
        pltpu.sync_copy(erel_hbm.at[pl.ds(base, CHUNK)], ridx)
        cp_s = pltpu.async_copy(ent_hbm.at[sidx], sbuf, sem_s)
        cp_d = pltpu.async_copy(ent_hbm.at[didx], dbuf, sem_d)
        cp_s.wait()
        cp_d.wait()

        def group_body(j, a):
          ea = list(a[0:nvec])
          ra = list(a[nvec:2 * nvec])
          wa = list(a[2 * nvec:3 * nvec])
          rvec = ridx[pl.ds(j * LANES, LANES)]
          for e in range(LANES):
            r = rvec[e]
            i = j * LANES + e
            for v in range(nvec):
              sl = pl.ds(v * LANES, LANES)
              rv = rtab[r, sl]
              sv = sbuf[i, sl]
              dv = dbuf[i, sl]
              ea[v] = ea[v] + jnp.maximum(sv + rv, 0.0)
              wa[v] = wa[v] + jnp.maximum(dv + rv, 0.0)
              ra[v] = ra[v] + rv
          return tuple(ea) + tuple(ra) + tuple(wa)

        return lax.fori_loop(0, CHUNK // LANES, group_body, acc)

      acc0 = (zvec,) * (3 * nvec)
      acc = lax.fori_loop(0, nchunk, chunk_body, acc0)
      for v in range(nvec):
        res[pl.ds(seg * 3 * H + v * LANES, LANES)] = acc[v] * inv_e
        res[pl.ds(seg * 3 * H + H + v * LANES, LANES)] = acc[nvec + v] * inv_e
        res[pl.ds(seg * 3 * H + 2 * H + v * LANES, LANES)] = (
            acc[2 * nvec + v] * inv_e)
      return carry

    lax.fori_loop(0, S, seg_body, 0)
    for s in range(S):
      dst_off = pl.multiple_of(s * (B * 3 * H) + wid * (3 * H), 3 * H)
      pltpu.sync_copy(res.at[pl.ds(s * 3 * H, 3 * H)],
                      out_hbm.at[pl.ds(dst_off, 3 * H)])

  return k(esrc, edst, erel, ent_tab, rel_tab_hbm)


def _tc_head(embed, W_ih, W_hh, bih, bhh, wr, br, prob, tl, S, B, H):
  def body(embed_ref, wih_ref, whh_ref, bih_ref, bhh_ref, wr_ref, br_ref,
           prob_ref, tl_ref, out_ref):
    h = jnp.zeros((B, H), jnp.float32)
    wih = wih_ref[...]
    whh = whh_ref[...]
    bih_v = bih_ref[...]
    bhh_v = bhh_ref[...]
    for s in range(S):
      x = embed_ref[s]
      gi = jnp.dot(x, wih, preferred_element_type=jnp.float32) + bih_v
      gh = jnp.dot(h, whh, preferred_element_type=jnp.float32) + bhh_v
      r = jax.nn.sigmoid(gi[:, 0:H] + gh[:, 0:H])
      z = jax.nn.sigmoid(gi[:, H:2 * H] + gh[:, H:2 * H])
      n = jnp.tanh(gi[:, 2 * H:3 * H] + r * gh[:, 2 * H:3 * H])
      h = (1.0 - z) * n + z * h
    logit = jnp.sum(h * wr_ref[...], axis=1, keepdims=True) + br_ref[0, 0]
    pred = jax.nn.sigmoid(logit)
    ii = lax.broadcasted_iota(jnp.int32, (B, prob_ref.shape[1]), 1)
    tmat = jnp.where(ii == tl_ref[...], prob_ref[...], 0.0)
    target = jnp.sum(tmat, axis=1, keepdims=True)
    eps = 1e-7
    p = jnp.clip(pred, eps, 1.0 - eps)
    li = target * jnp.log(p) + (1.0 - target) * jnp.log(1.0 - p)
    out_ref[...] = jnp.reshape(-jnp.mean(li), (1, 1))

  return pl.pallas_call(
      body,
      out_shape=jax.ShapeDtypeStruct((1, 1), jnp.float32),
  )(embed, W_ih, W_hh, bih, bhh, wr, br, prob, tl)


def kernel(t_list, true_prob_r, edge_src, edge_dst, edge_rel,
           ent_embeds, rel_embeds, W_ih, W_hh, b_ih, b_hh, W_r, b_r):
  B, S, E = edge_src.shape
  H = ent_embeds.shape[1]
  num_ents = ent_embeds.shape[0]
  num_rels = rel_embeds.shape[0]
  EP = ((E + CHUNK - 1) // CHUNK) * CHUNK

  ent2 = jnp.concatenate(
      [ent_embeds, jnp.zeros((8, H), jnp.float32)], axis=0)
  rel2 = jnp.concatenate(
      [rel_embeds, jnp.zeros((8, H), jnp.float32)], axis=0)

  def pad_edges(e, fill):
    e2 = e.reshape(B * S, E).astype(jnp.int32)
    pad = jnp.full((B * S, EP - E), fill, jnp.int32)
    return jnp.concatenate([e2, pad], axis=1).reshape(-1)

  esrc = pad_edges(edge_src, num_ents)
  edst = pad_edges(edge_dst, num_ents)
  erel = pad_edges(edge_rel, num_rels)

  embed_flat = _sc_aggregate(esrc, edst, erel, ent2, rel2, S, B, H, EP, E)
  embed = embed_flat.reshape(S, B, 3 * H)

  T = true_prob_r.shape[0]
  TP = ((T + H - 1) // H) * H
  prob = jnp.concatenate(
      [true_prob_r, jnp.zeros((TP - T,), jnp.float32)]).reshape(1, TP)
  tl = t_list.astype(jnp.int32).reshape(B, 1)

  loss = _tc_head(embed, W_ih, W_hh,
                  b_ih.reshape(1, 3 * H), b_hh.reshape(1, 3 * H),
                  W_r.reshape(1, H), b_r.reshape(1, 1),
                  prob, tl, S, B, H)
  return loss[0, 0]


# speedup vs baseline: 6.2509x; 1.0770x over previous
import functools

import jax
import jax.numpy as jnp
from jax import lax
from jax.experimental import pallas as pl
from jax.experimental.pallas import tpu as pltpu
from jax.experimental.pallas import tpu_sc as plsc

NC = 2
NS = 16
NW = NC * NS
LANES = 16
CHUNK = 128


def _sc_aggregate(eall, ent_tab, rel_tab_hbm, S, B, H, EP, E):
  nchunk = EP // CHUNK
  nvec = H // LANES
  rel_rows = rel_tab_hbm.shape[0]
  inv_e = 1.0 / float(E)
  idxseg = nchunk * 3 * CHUNK
  mesh = plsc.VectorSubcoreMesh(core_axis_name="c", subcore_axis_name="s")

  @functools.partial(
      pl.kernel,
      out_type=jax.ShapeDtypeStruct((S * B * 3 * H,), jnp.float32),
      mesh=mesh,
      scratch_types=[
          pltpu.VMEM((rel_rows, H), jnp.float32),
          pltpu.VMEM((idxseg,), jnp.int32),
          pltpu.VMEM((CHUNK, H), jnp.float32),
          pltpu.VMEM((CHUNK, H), jnp.float32),
          pltpu.VMEM((CHUNK, H), jnp.float32),
          pltpu.VMEM((CHUNK, H), jnp.float32),
          pltpu.VMEM((S * 3 * H,), jnp.float32),
          pltpu.SemaphoreType.DMA,
          pltpu.SemaphoreType.DMA,
      ],
  )
  def k(eall_hbm, ent_hbm, rel_hbm, out_hbm,
        rtab, idxb, sbufP, dbufP, sbufQ, dbufQ, res, semP, semQ):
    wid = lax.axis_index("s") * NC + lax.axis_index("c")
    pltpu.sync_copy(rel_hbm, rtab)
    zvec = jnp.zeros((LANES,), jnp.float32)

    def gather(c, sb, db, sem):
      coff = c * (3 * CHUNK)
      pltpu.async_copy(ent_hbm.at[idxb.at[pl.ds(coff, CHUNK)]], sb, sem)
      pltpu.async_copy(
          ent_hbm.at[idxb.at[pl.ds(coff + CHUNK, CHUNK)]], db, sem)

    def wait2(sb, db, sem):
      pltpu.make_async_copy(ent_hbm.at[pl.ds(0, CHUNK)], sb, sem).wait()
      pltpu.make_async_copy(ent_hbm.at[pl.ds(0, CHUNK)], db, sem).wait()

    def compute(c, sb, db, acc):
      def group_body(j, a):
        ea = list(a[0:nvec])
        ra = list(a[nvec:2 * nvec])
        wa = list(a[2 * nvec:3 * nvec])
        rvec = idxb[pl.ds(c * (3 * CHUNK) + 2 * CHUNK + j * LANES, LANES)]
        for e in range(LANES):
          r = rvec[e]
          i = j * LANES + e
          for v in range(nvec):
            sl = pl.ds(v * LANES, LANES)
            rv = rtab[r, sl]
            sv = sb[i, sl]
            dv = db[i, sl]
            ea[v] = ea[v] + jnp.maximum(sv + rv, 0.0)
            wa[v] = wa[v] + jnp.maximum(dv + rv, 0.0)
            ra[v] = ra[v] + rv
        return tuple(ea) + tuple(ra) + tuple(wa)

      return lax.fori_loop(0, CHUNK // LANES, group_body, acc)

    def seg_body(seg, carry):
      ibase = pl.multiple_of((wid * S + seg) * idxseg, CHUNK)
      pltpu.sync_copy(eall_hbm.at[pl.ds(ibase, idxseg)], idxb)
      gather(0, sbufP, dbufP, semP)

      def pair_body(cp, acc):
        c0 = cp * 2
        c1 = c0 + 1
        gather(c1, sbufQ, dbufQ, semQ)
        wait2(sbufP, dbufP, semP)
        acc = compute(c0, sbufP, dbufP, acc)

        @pl.when(c1 + 1 < nchunk)
        def _():
          gather(c1 + 1, sbufP, dbufP, semP)

        wait2(sbufQ, dbufQ, semQ)
        return compute(c1, sbufQ, dbufQ, acc)

      acc0 = (zvec,) * (3 * nvec)
      acc = lax.fori_loop(0, nchunk // 2, pair_body, acc0)
      for v in range(nvec):
        res[pl.ds(seg * 3 * H + v * LANES, LANES)] = acc[v] * inv_e
        res[pl.ds(seg * 3 * H + H + v * LANES, LANES)] = acc[nvec + v] * inv_e
        res[pl.ds(seg * 3 * H + 2 * H + v * LANES, LANES)] = (
            acc[2 * nvec + v] * inv_e)
      return carry

    lax.fori_loop(0, S, seg_body, 0)
    for s in range(S):
      dst_off = pl.multiple_of(s * (B * 3 * H) + wid * (3 * H), 3 * H)
      pltpu.sync_copy(res.at[pl.ds(s * 3 * H, 3 * H)],
                      out_hbm.at[pl.ds(dst_off, 3 * H)])

  return k(eall, ent_tab, rel_tab_hbm)


def _tc_head(embed, W_ih, W_hh, bih, bhh, wr, br, prob, tl, S, B, H):
  def body(embed_ref, wih_ref, whh_ref, bih_ref, bhh_ref, wr_ref, br_ref,
           prob_ref, tl_ref, out_ref):
    h = jnp.zeros((B, H), jnp.float32)
    wih = wih_ref[...]
    whh = whh_ref[...]
    bih_v = bih_ref[...]
    bhh_v = bhh_ref[...]
    for s in range(S):
      x = embed_ref[s]
      gi = jnp.dot(x, wih, preferred_element_type=jnp.float32) + bih_v
      gh = jnp.dot(h, whh, preferred_element_type=jnp.float32) + bhh_v
      r = jax.nn.sigmoid(gi[:, 0:H] + gh[:, 0:H])
      z = jax.nn.sigmoid(gi[:, H:2 * H] + gh[:, H:2 * H])
      n = jnp.tanh(gi[:, 2 * H:3 * H] + r * gh[:, 2 * H:3 * H])
      h = (1.0 - z) * n + z * h
    logit = jnp.sum(h * wr_ref[...], axis=1, keepdims=True) + br_ref[0, 0]
    pred = jax.nn.sigmoid(logit)
    ii = lax.broadcasted_iota(jnp.int32, (B, prob_ref.shape[1]), 1)
    tmat = jnp.where(ii == tl_ref[...], prob_ref[...], 0.0)
    target = jnp.sum(tmat, axis=1, keepdims=True)
    eps = 1e-7
    p = jnp.clip(pred, eps, 1.0 - eps)
    li = target * jnp.log(p) + (1.0 - target) * jnp.log(1.0 - p)
    out_ref[...] = jnp.reshape(-jnp.mean(li), (1, 1))

  return pl.pallas_call(
      body,
      out_shape=jax.ShapeDtypeStruct((1, 1), jnp.float32),
  )(embed, W_ih, W_hh, bih, bhh, wr, br, prob, tl)


def kernel(t_list, true_prob_r, edge_src, edge_dst, edge_rel,
           ent_embeds, rel_embeds, W_ih, W_hh, b_ih, b_hh, W_r, b_r):
  B, S, E = edge_src.shape
  H = ent_embeds.shape[1]
  num_ents = ent_embeds.shape[0]
  num_rels = rel_embeds.shape[0]
  EP = ((E + CHUNK - 1) // CHUNK) * CHUNK

  ent2 = jnp.concatenate(
      [ent_embeds, jnp.zeros((8, H), jnp.float32)], axis=0)
  rel2 = jnp.concatenate(
      [rel_embeds, jnp.zeros((8, H), jnp.float32)], axis=0)

  nchunk = EP // CHUNK

  def pad_edges(e, fill):
    e2 = e.reshape(B * S, E).astype(jnp.int32)
    pad = jnp.full((B * S, EP - E), fill, jnp.int32)
    return jnp.concatenate([e2, pad], axis=1).reshape(B * S, nchunk, CHUNK)

  esrc = pad_edges(edge_src, num_ents)
  edst = pad_edges(edge_dst, num_ents)
  erel = pad_edges(edge_rel, num_rels)
  eall = jnp.stack([esrc, edst, erel], axis=2).reshape(-1)

  embed_flat = _sc_aggregate(eall, ent2, rel2, S, B, H, EP, E)
  embed = embed_flat.reshape(S, B, 3 * H)

  T = true_prob_r.shape[0]
  TP = ((T + H - 1) // H) * H
  prob = jnp.concatenate(
      [true_prob_r, jnp.zeros((TP - T,), jnp.float32)]).reshape(1, TP)
  tl = t_list.astype(jnp.int32).reshape(B, 1)

  loss = _tc_head(embed, W_ih, W_hh,
                  b_ih.reshape(1, 3 * H), b_hh.reshape(1, 3 * H),
                  W_r.reshape(1, H), b_r.reshape(1, 1),
                  prob, tl, S, B, H)
  return loss[0, 0]
